# baseline (speedup 1.0000x reference)
import functools
import numpy as np
import jax
import jax.numpy as jnp
from jax import lax
from jax.experimental import pallas as pl
from jax.experimental.pallas import tpu as pltpu
from jax.experimental.pallas import tpu_sc as plsc

N = 10000
E = 320000
D = 128
AVG_LOG_C = float(np.log(33.0))

NBLK1 = 2000
MBLK = 4000
EBLK = 512
NBLK4 = 400
NWORK = 32
EPW = E // NWORK
EPT = E // 16
CH = 200
ROWS_PT = N // 16
ROWS_ALN = 624
ROWS_TAIL = N - 16 * ROWS_ALN

_PREC = lax.Precision.HIGHEST


def _pre_body(x_ref, wd_ref, ws_ref, b_ref, a_ref, bo_ref):
    xb = x_ref[...]
    a_ref[...] = jnp.dot(xb, wd_ref[...], precision=_PREC,
                         preferred_element_type=jnp.float32) + b_ref[...]
    bo_ref[...] = jnp.dot(xb, ws_ref[...], precision=_PREC,
                          preferred_element_type=jnp.float32)


def _pre_call(x, wd, ws, b):
    return pl.pallas_call(
        _pre_body,
        grid=(N // NBLK1,),
        in_specs=[
            pl.BlockSpec((NBLK1, D), lambda i: (i, 0)),
            pl.BlockSpec((D, D), lambda i: (0, 0)),
            pl.BlockSpec((D, D), lambda i: (0, 0)),
            pl.BlockSpec((1, D), lambda i: (0, 0)),
        ],
        out_specs=[
            pl.BlockSpec((NBLK1, D), lambda i: (i, 0)),
            pl.BlockSpec((NBLK1, D), lambda i: (i, 0)),
        ],
        out_shape=[jax.ShapeDtypeStruct((N, D), jnp.float32)] * 2,
    )(x, wd, ws, b)


def _sc_gather_body(a_hbm, b_hbm, dst_hbm, src_hbm, oa_hbm, ob_hbm,
                    dst_v, src_v, bufa, bufb, sema, semb):
    wid = lax.axis_index("s") * 2 + lax.axis_index("c")
    base = wid * EPW
    pltpu.sync_copy(dst_hbm.at[pl.ds(base, EPW)], dst_v)
    pltpu.sync_copy(src_hbm.at[pl.ds(base, EPW)], src_v)

    def chunk(ci, carry):
        off = ci * CH
        ca = pltpu.async_copy(a_hbm.at[dst_v.at[pl.ds(off, CH)]], bufa, sema)
        cb = pltpu.async_copy(b_hbm.at[src_v.at[pl.ds(off, CH)]], bufb, semb)
        ca.wait()
        cb.wait()
        pltpu.sync_copy(bufa, oa_hbm.at[pl.ds(base + off, CH)])
        pltpu.sync_copy(bufb, ob_hbm.at[pl.ds(base + off, CH)])
        return carry

    lax.fori_loop(0, EPW // CH, chunk, 0)


_sc_gather = functools.partial(
    pl.kernel,
    mesh=plsc.VectorSubcoreMesh(core_axis_name="c", subcore_axis_name="s"),
    out_type=[jax.ShapeDtypeStruct((E, D), jnp.float32)] * 2,
    scratch_types=[
        pltpu.VMEM((EPW,), jnp.int32),
        pltpu.VMEM((EPW,), jnp.int32),
        pltpu.VMEM((CH, D), jnp.float32),
        pltpu.VMEM((CH, D), jnp.float32),
        pltpu.SemaphoreType.DMA,
        pltpu.SemaphoreType.DMA,
    ],
)(_sc_gather_body)


def _msg_body(ga_ref, gb_ref, m_ref, m2_ref):
    m = ga_ref[...] + gb_ref[...]
    m_ref[...] = m
    m2_ref[...] = m * m


def _msg_call(ga, gb):
    blk = pl.BlockSpec((MBLK, D), lambda i: (i, 0))
    return pl.pallas_call(
        _msg_body,
        grid=(E // MBLK,),
        in_specs=[blk, blk],
        out_specs=[blk, blk],
        out_shape=[jax.ShapeDtypeStruct((E, D), jnp.float32)] * 2,
    )(ga, gb)


def _sc_scatter_body(m_hbm, m2_hbm, dst_hbm, z128_hbm,
                     s_out, ss_out,
                     idx_v, buf, acc_sh):
    cid = lax.axis_index("c")
    sid = lax.axis_index("s")

    @pl.when(sid == 0)
    def _init():
        pltpu.sync_copy(z128_hbm, acc_sh)

    plsc.subcore_barrier()

    base = sid * EPT

    def chunk(ci, carry):
        off = base + ci * CH
        pltpu.sync_copy(dst_hbm.at[pl.ds(off, CH)], idx_v)

        @pl.when(cid == 0)
        def _c0():
            pltpu.sync_copy(m_hbm.at[pl.ds(off, CH)], buf)
            pltpu.sync_copy(buf, acc_sh.at[idx_v], add=True)

        @pl.when(cid == 1)
        def _c1():
            pltpu.sync_copy(m2_hbm.at[pl.ds(off, CH)], buf)
            pltpu.sync_copy(buf, acc_sh.at[idx_v], add=True)

        return carry

    lax.fori_loop(0, EPT // CH, chunk, 0)
    plsc.subcore_barrier()

    rbase = sid * ROWS_ALN

    @pl.when(cid == 0)
    def _out0():
        pltpu.sync_copy(acc_sh.at[pl.ds(rbase, ROWS_ALN)],
                        s_out.at[pl.ds(rbase, ROWS_ALN)])

        @pl.when(sid == 15)
        def _tail0():
            pltpu.sync_copy(acc_sh.at[pl.ds(16 * ROWS_ALN, ROWS_TAIL)],
                            s_out.at[pl.ds(16 * ROWS_ALN, ROWS_TAIL)])

    @pl.when(cid == 1)
    def _out1():
        pltpu.sync_copy(acc_sh.at[pl.ds(rbase, ROWS_ALN)],
                        ss_out.at[pl.ds(rbase, ROWS_ALN)])

        @pl.when(sid == 15)
        def _tail1():
            pltpu.sync_copy(acc_sh.at[pl.ds(16 * ROWS_ALN, ROWS_TAIL)],
                            ss_out.at[pl.ds(16 * ROWS_ALN, ROWS_TAIL)])


_sc_scatter = functools.partial(
    pl.kernel,
    mesh=plsc.VectorSubcoreMesh(core_axis_name="c", subcore_axis_name="s"),
    out_type=[
        jax.ShapeDtypeStruct((N, D), jnp.float32),
        jax.ShapeDtypeStruct((N, D), jnp.float32),
    ],
    scratch_types=[
        pltpu.VMEM((CH,), jnp.int32),
        pltpu.VMEM((CH, D), jnp.float32),
        pltpu.MemorySpace.VMEM_SHARED((N, D), jnp.float32),
    ],
)(_sc_scatter_body)


def _seg_body(dst_ref, m_ref, mn_ref, mx_ref, cnt_ref):
    @pl.when(pl.program_id(0) == 0)
    def _init():
        mn_ref[...] = jnp.full((N, D), 3.4e38, jnp.float32)
        mx_ref[...] = jnp.full((N, D), -3.4e38, jnp.float32)
        cnt_ref[...] = jnp.zeros((N, D), jnp.float32)

    def body(e, carry):
        d = dst_ref[e]
        m = m_ref[pl.ds(e, 1), :]
        rs = pl.ds(d, 1)
        mn_ref[rs, :] = jnp.minimum(mn_ref[rs, :], m)
        mx_ref[rs, :] = jnp.maximum(mx_ref[rs, :], m)
        cnt_ref[rs, :] = cnt_ref[rs, :] + 1.0
        return carry

    lax.fori_loop(0, EBLK, body, 0)


def _seg_call(dst, m):
    return pl.pallas_call(
        _seg_body,
        grid=(E // EBLK,),
        in_specs=[
            pl.BlockSpec((EBLK,), lambda i: (i,), memory_space=pltpu.SMEM),
            pl.BlockSpec((EBLK, D), lambda i: (i, 0)),
        ],
        out_specs=[pl.BlockSpec((N, D), lambda i: (0, 0))] * 3,
        out_shape=[jax.ShapeDtypeStruct((N, D), jnp.float32)] * 3,
    )(dst, m)


def _post_body(x_ref, s_ref, ss_ref, mn_ref, mx_ref, cnt_ref,
               wp_ref, bp_ref, wl_ref, bl_ref, o_ref):
    xb = x_ref[...]
    c = cnt_ref[...][:, :1]
    denom = jnp.maximum(c, 1.0)
    mean = s_ref[...] / denom
    msq = ss_ref[...] / denom
    var = msq - mean * mean
    std = jnp.sqrt(jnp.maximum(var, 0.0) + 1e-5)
    has = c > 0.0
    mn = jnp.where(has, mn_ref[...], 0.0)
    mx = jnp.where(has, mx_ref[...], 0.0)
    agg = jnp.concatenate([mean, mn, mx, std], axis=1)
    lg = jnp.log(denom + 1.0)
    amp = agg * (lg / AVG_LOG_C)
    att = agg * (AVG_LOG_C / lg)
    full = jnp.concatenate([xb, agg, amp, att], axis=1)
    out = jnp.dot(full, wp_ref[...], precision=_PREC,
                  preferred_element_type=jnp.float32) + bp_ref[...]
    out = jnp.dot(out, wl_ref[...], precision=_PREC,
                  preferred_element_type=jnp.float32) + bl_ref[...]
    o_ref[...] = xb + out


def _post_call(x, s, ss, mn, mx, cnt, wp, bp, wl, bl):
    nd = pl.BlockSpec((NBLK4, D), lambda i: (i, 0))
    return pl.pallas_call(
        _post_body,
        grid=(N // NBLK4,),
        in_specs=[
            nd, nd, nd, nd, nd,
            nd,
            pl.BlockSpec((13 * D, D), lambda i: (0, 0)),
            pl.BlockSpec((1, D), lambda i: (0, 0)),
            pl.BlockSpec((D, D), lambda i: (0, 0)),
            pl.BlockSpec((1, D), lambda i: (0, 0)),
        ],
        out_specs=pl.BlockSpec((NBLK4, D), lambda i: (i, 0)),
        out_shape=jax.ShapeDtypeStruct((N, D), jnp.float32),
    )(x, s, ss, mn, mx, cnt, wp, bp, wl, bl)


def kernel(x, edge_index, W_pre, b_pre, W_post, b_post, W_lin, b_lin):
    src = edge_index[0]
    dst = edge_index[1]
    a, b = _pre_call(x, W_pre[:D], W_pre[D:], b_pre.reshape(1, D))
    ga, gb = _sc_gather(a, b, dst, src)
    m, m2 = _msg_call(ga, gb)
    z128 = jnp.zeros((N, D), jnp.float32)
    s, ss = _sc_scatter(m, m2, dst, z128)
    mn, mx, cnt = _seg_call(dst, m)
    return _post_call(x, s, ss, mn, mx, cnt, W_post, b_post.reshape(1, D),
                      W_lin, b_lin.reshape(1, D))


# speedup vs baseline: 1.5316x; 1.5316x over previous
import functools
import numpy as np
import jax
import jax.numpy as jnp
from jax import lax
from jax.experimental import pallas as pl
from jax.experimental.pallas import tpu as pltpu
from jax.experimental.pallas import tpu_sc as plsc

N = 10000
E = 320000
D = 128
AVG_LOG_C = float(np.log(33.0))

NBLK1 = 2000
MBLK = 4000
EBLK = 512
NBLK4 = 400
NWORK = 32
EPW = E // NWORK
EPT = E // 16
CH = 200
ROWS_PT = N // 16
ROWS_ALN = 624
ROWS_TAIL = N - 16 * ROWS_ALN

_PREC = lax.Precision.HIGHEST


def _pre_body(x_ref, wd_ref, ws_ref, b_ref, a_ref, bo_ref):
    xb = x_ref[...]
    a_ref[...] = jnp.dot(xb, wd_ref[...], precision=_PREC,
                         preferred_element_type=jnp.float32) + b_ref[...]
    bo_ref[...] = jnp.dot(xb, ws_ref[...], precision=_PREC,
                          preferred_element_type=jnp.float32)


def _pre_call(x, wd, ws, b):
    return pl.pallas_call(
        _pre_body,
        grid=(N // NBLK1,),
        in_specs=[
            pl.BlockSpec((NBLK1, D), lambda i: (i, 0)),
            pl.BlockSpec((D, D), lambda i: (0, 0)),
            pl.BlockSpec((D, D), lambda i: (0, 0)),
            pl.BlockSpec((1, D), lambda i: (0, 0)),
        ],
        out_specs=[
            pl.BlockSpec((NBLK1, D), lambda i: (i, 0)),
            pl.BlockSpec((NBLK1, D), lambda i: (i, 0)),
        ],
        out_shape=[jax.ShapeDtypeStruct((N, D), jnp.float32)] * 2,
    )(x, wd, ws, b)


def _sc_gather_body(a_hbm, b_hbm, dst_hbm, src_hbm, oa_hbm, ob_hbm,
                    dst_v, src_v, bufa, bufb, sema, semb):
    wid = lax.axis_index("s") * 2 + lax.axis_index("c")
    base = wid * EPW
    pltpu.sync_copy(dst_hbm.at[pl.ds(base, EPW)], dst_v)
    pltpu.sync_copy(src_hbm.at[pl.ds(base, EPW)], src_v)

    def chunk(ci, carry):
        off = ci * CH
        ca = pltpu.async_copy(a_hbm.at[dst_v.at[pl.ds(off, CH)]], bufa, sema)
        cb = pltpu.async_copy(b_hbm.at[src_v.at[pl.ds(off, CH)]], bufb, semb)
        ca.wait()
        cb.wait()
        pltpu.sync_copy(bufa, oa_hbm.at[pl.ds(base + off, CH)])
        pltpu.sync_copy(bufb, ob_hbm.at[pl.ds(base + off, CH)])
        return carry

    lax.fori_loop(0, EPW // CH, chunk, 0)


_sc_gather = functools.partial(
    pl.kernel,
    mesh=plsc.VectorSubcoreMesh(core_axis_name="c", subcore_axis_name="s"),
    out_type=[jax.ShapeDtypeStruct((E, D), jnp.float32)] * 2,
    scratch_types=[
        pltpu.VMEM((EPW,), jnp.int32),
        pltpu.VMEM((EPW,), jnp.int32),
        pltpu.VMEM((CH, D), jnp.float32),
        pltpu.VMEM((CH, D), jnp.float32),
        pltpu.SemaphoreType.DMA,
        pltpu.SemaphoreType.DMA,
    ],
)(_sc_gather_body)


def _msg_body(ga_ref, gb_ref, m_ref, m2_ref):
    m = ga_ref[...] + gb_ref[...]
    m_ref[...] = m
    m2_ref[...] = m * m


def _msg_call(ga, gb):
    blk = pl.BlockSpec((MBLK, D), lambda i: (i, 0))
    return pl.pallas_call(
        _msg_body,
        grid=(E // MBLK,),
        in_specs=[blk, blk],
        out_specs=[blk, blk],
        out_shape=[jax.ShapeDtypeStruct((E, D), jnp.float32)] * 2,
    )(ga, gb)


def _sc_scatter_body(m_hbm, m2_hbm, dst_hbm, z128_hbm,
                     s_out, ss_out,
                     idx_v, buf, acc_sh):
    cid = lax.axis_index("c")
    sid = lax.axis_index("s")

    @pl.when(sid == 0)
    def _init():
        pltpu.sync_copy(z128_hbm, acc_sh)

    plsc.subcore_barrier()

    base = sid * EPT

    def chunk(ci, carry):
        off = base + ci * CH
        pltpu.sync_copy(dst_hbm.at[pl.ds(off, CH)], idx_v)

        @pl.when(cid == 0)
        def _c0():
            pltpu.sync_copy(m_hbm.at[pl.ds(off, CH)], buf)
            pltpu.sync_copy(buf, acc_sh.at[idx_v], add=True)

        @pl.when(cid == 1)
        def _c1():
            pltpu.sync_copy(m2_hbm.at[pl.ds(off, CH)], buf)
            pltpu.sync_copy(buf, acc_sh.at[idx_v], add=True)

        return carry

    lax.fori_loop(0, EPT // CH, chunk, 0)
    plsc.subcore_barrier()

    rbase = sid * ROWS_ALN

    @pl.when(cid == 0)
    def _out0():
        pltpu.sync_copy(acc_sh.at[pl.ds(rbase, ROWS_ALN)],
                        s_out.at[pl.ds(rbase, ROWS_ALN)])

        @pl.when(sid == 15)
        def _tail0():
            pltpu.sync_copy(acc_sh.at[pl.ds(16 * ROWS_ALN, ROWS_TAIL)],
                            s_out.at[pl.ds(16 * ROWS_ALN, ROWS_TAIL)])

    @pl.when(cid == 1)
    def _out1():
        pltpu.sync_copy(acc_sh.at[pl.ds(rbase, ROWS_ALN)],
                        ss_out.at[pl.ds(rbase, ROWS_ALN)])

        @pl.when(sid == 15)
        def _tail1():
            pltpu.sync_copy(acc_sh.at[pl.ds(16 * ROWS_ALN, ROWS_TAIL)],
                            ss_out.at[pl.ds(16 * ROWS_ALN, ROWS_TAIL)])


_sc_scatter = functools.partial(
    pl.kernel,
    mesh=plsc.VectorSubcoreMesh(core_axis_name="c", subcore_axis_name="s"),
    out_type=[
        jax.ShapeDtypeStruct((N, D), jnp.float32),
        jax.ShapeDtypeStruct((N, D), jnp.float32),
    ],
    scratch_types=[
        pltpu.VMEM((CH,), jnp.int32),
        pltpu.VMEM((CH, D), jnp.float32),
        pltpu.MemorySpace.VMEM_SHARED((N, D), jnp.float32),
    ],
)(_sc_scatter_body)


def _seg_body(dst_ref, m_ref, mn_ref, mx_ref, cnt_ref):
    @pl.when(pl.program_id(0) == 0)
    def _init():
        mn_ref[...] = jnp.full((N, D), 3.4e38, jnp.float32)
        mx_ref[...] = jnp.full((N, D), -3.4e38, jnp.float32)
        cnt_ref[...] = jnp.zeros((N, D), jnp.float32)

    def body(e, carry):
        d = dst_ref[e]
        m = m_ref[pl.ds(e, 1), :]
        rs = pl.ds(d, 1)
        mn_ref[rs, :] = jnp.minimum(mn_ref[rs, :], m)
        mx_ref[rs, :] = jnp.maximum(mx_ref[rs, :], m)
        cnt_ref[rs, :] = cnt_ref[rs, :] + 1.0
        return carry

    lax.fori_loop(0, EBLK, body, 0, unroll=8)


def _seg_call(dst, m):
    return pl.pallas_call(
        _seg_body,
        grid=(E // EBLK,),
        in_specs=[
            pl.BlockSpec((EBLK,), lambda i: (i,), memory_space=pltpu.SMEM),
            pl.BlockSpec((EBLK, D), lambda i: (i, 0)),
        ],
        out_specs=[pl.BlockSpec((N, D), lambda i: (0, 0))] * 3,
        out_shape=[jax.ShapeDtypeStruct((N, D), jnp.float32)] * 3,
    )(dst, m)


def _post_body(x_ref, s_ref, ss_ref, mn_ref, mx_ref, cnt_ref,
               wp_ref, bp_ref, wl_ref, bl_ref, o_ref):
    xb = x_ref[...]
    c = cnt_ref[...][:, :1]
    denom = jnp.maximum(c, 1.0)
    mean = s_ref[...] / denom
    msq = ss_ref[...] / denom
    var = msq - mean * mean
    std = jnp.sqrt(jnp.maximum(var, 0.0) + 1e-5)
    has = c > 0.0
    mn = jnp.where(has, mn_ref[...], 0.0)
    mx = jnp.where(has, mx_ref[...], 0.0)
    agg = jnp.concatenate([mean, mn, mx, std], axis=1)
    lg = jnp.log(denom + 1.0)
    amp = agg * (lg / AVG_LOG_C)
    att = agg * (AVG_LOG_C / lg)
    full = jnp.concatenate([xb, agg, amp, att], axis=1)
    out = jnp.dot(full, wp_ref[...], precision=_PREC,
                  preferred_element_type=jnp.float32) + bp_ref[...]
    out = jnp.dot(out, wl_ref[...], precision=_PREC,
                  preferred_element_type=jnp.float32) + bl_ref[...]
    o_ref[...] = xb + out


def _post_call(x, s, ss, mn, mx, cnt, wp, bp, wl, bl):
    nd = pl.BlockSpec((NBLK4, D), lambda i: (i, 0))
    return pl.pallas_call(
        _post_body,
        grid=(N // NBLK4,),
        in_specs=[
            nd, nd, nd, nd, nd,
            nd,
            pl.BlockSpec((13 * D, D), lambda i: (0, 0)),
            pl.BlockSpec((1, D), lambda i: (0, 0)),
            pl.BlockSpec((D, D), lambda i: (0, 0)),
            pl.BlockSpec((1, D), lambda i: (0, 0)),
        ],
        out_specs=pl.BlockSpec((NBLK4, D), lambda i: (i, 0)),
        out_shape=jax.ShapeDtypeStruct((N, D), jnp.float32),
    )(x, s, ss, mn, mx, cnt, wp, bp, wl, bl)


def kernel(x, edge_index, W_pre, b_pre, W_post, b_post, W_lin, b_lin):
    src = edge_index[0]
    dst = edge_index[1]
    a, b = _pre_call(x, W_pre[:D], W_pre[D:], b_pre.reshape(1, D))
    ga, gb = _sc_gather(a, b, dst, src)
    m, m2 = _msg_call(ga, gb)
    z128 = jnp.zeros((N, D), jnp.float32)
    s, ss = _sc_scatter(m, m2, dst, z128)
    mn, mx, cnt = _seg_call(dst, m)
    return _post_call(x, s, ss, mn, mx, cnt, W_post, b_post.reshape(1, D),
                      W_lin, b_lin.reshape(1, D))
